# Initial kernel scaffold; baseline (speedup 1.0000x reference)
import functools

import jax
import jax.numpy as jnp
from jax import lax
from jax.experimental import pallas as pl
from jax.experimental.pallas import tpu as pltpu
from jax.experimental.pallas import tpu_sc as plsc

N = 100000
M = 4096
W = 16
L = 16
CHUNK = 6256
BASE = 6248
NV = CHUNK // L
TBL = 3 * M
SLICE = TBL // W

_mesh = plsc.VectorSubcoreMesh(
    core_axis_name="c", subcore_axis_name="s", num_cores=1
)


@functools.partial(
    pl.kernel,
    out_type=jax.ShapeDtypeStruct((N,), jnp.float32),
    mesh=_mesh,
    scratch_types=[
        pltpu.VMEM((CHUNK,), jnp.int32),
        pltpu.VMEM((2 * CHUNK,), jnp.float32),
        pltpu.VMEM((CHUNK,), jnp.float32),
        pltpu.VMEM((CHUNK,), jnp.float32),
        pltpu.VMEM((CHUNK,), jnp.float32),
        pltpu.VMEM((TBL,), jnp.float32),
        pltpu.VMEM((W, SLICE), jnp.float32),
        pltpu.VMEM((SLICE,), jnp.float32),
        pltpu.VMEM((M,), jnp.float32),
        pltpu.VMEM_SHARED((W, TBL), jnp.float32),
        pltpu.VMEM_SHARED((TBL,), jnp.float32),
    ],
)
def _ce_kernel(ids_hbm, h_hbm, q_hbm, out_hbm,
               ids_v, h_v, q_v, sinv_v, out_v, tbl_v, part_v, comb_v, r_v,
               sh_all, sh_comb):
    w = lax.axis_index("s")
    start = BASE * w + 8 * jnp.maximum(w - 12, 0)
    cnt = jnp.where(w >= 12, CHUNK, BASE)

    pltpu.sync_copy(ids_hbm.at[pl.ds(start, CHUNK)], ids_v)
    pltpu.sync_copy(h_hbm.at[pl.ds(2 * start, 2 * CHUNK)], h_v)
    pltpu.sync_copy(q_hbm.at[pl.ds(start, CHUNK)], q_v)

    lanes = lax.iota(jnp.int32, L)
    zero = jnp.zeros((L,), jnp.float32)

    def zero_body(k, c):
        tbl_v[pl.ds(k * L, L)] = zero
        return c

    lax.fori_loop(0, TBL // L, zero_body, 0)

    def p1(j, c):
        base = j * L
        ii2 = 2 * base + 2 * lanes
        idx = ids_v[pl.ds(base, L)]
        e = plsc.load_gather(h_v, [ii2])
        s = plsc.load_gather(h_v, [ii2 + 1])
        qv = q_v[pl.ds(base, L)]
        sinv = 1.0 / s
        esinv = e * sinv
        sinv_v[pl.ds(base, L)] = sinv
        m = (base + lanes) < cnt
        plsc.addupdate_scatter(tbl_v, [idx], qv, mask=m)
        plsc.addupdate_scatter(tbl_v, [idx + M], sinv, mask=m)
        plsc.addupdate_scatter(tbl_v, [idx + 2 * M], esinv, mask=m)
        return c

    lax.fori_loop(0, NV, p1, 0)

    pltpu.sync_copy(tbl_v, sh_all.at[w])
    plsc.subcore_barrier()
    pltpu.sync_copy(sh_all.at[:, pl.ds(SLICE * w, SLICE)], part_v)

    def red_body(r, c):
        o = r * L
        acc = part_v[0, pl.ds(o, L)]
        for t in range(1, W):
            acc = acc + part_v[t, pl.ds(o, L)]
        comb_v[pl.ds(o, L)] = acc
        return c

    lax.fori_loop(0, SLICE // L, red_body, 0)
    pltpu.sync_copy(comb_v, sh_comb.at[pl.ds(SLICE * w, SLICE)])
    plsc.subcore_barrier()
    pltpu.sync_copy(sh_comb, tbl_v)

    def rt(k, c):
        o = k * L
        tq = tbl_v[pl.ds(o, L)]
        tsinv = tbl_v[pl.ds(M + o, L)]
        tesinv = tbl_v[pl.ds(2 * M + o, L)]
        r_v[pl.ds(o, L)] = (tq + tesinv) / tsinv
        return c

    lax.fori_loop(0, M // L, rt, 0)

    def p2(j, c):
        base = j * L
        ii2 = 2 * base + 2 * lanes
        idx = ids_v[pl.ds(base, L)]
        e = plsc.load_gather(h_v, [ii2])
        sinv = sinv_v[pl.ds(base, L)]
        r = plsc.load_gather(r_v, [idx])
        out_v[pl.ds(base, L)] = sinv * (r - e)
        return c

    lax.fori_loop(0, NV, p2, 0)

    pltpu.sync_copy(out_v, out_hbm.at[pl.ds(start, CHUNK)])


def kernel(h, q, mol_ids):
    ids32 = mol_ids.astype(jnp.int32)
    return _ce_kernel(ids32, h.reshape(-1), q)


# speedup vs baseline: 15.5653x; 15.5653x over previous
import functools

import jax
import jax.numpy as jnp
from jax import lax
from jax.experimental import pallas as pl
from jax.experimental.pallas import tpu as pltpu
from jax.experimental.pallas import tpu_sc as plsc

N = 100000
M = 4096
W = 16
L = 16
CHUNK = 6256
BASE = 6248
NV = CHUNK // L
TBL = 3 * M
SLICE = TBL // W

_mesh = plsc.VectorSubcoreMesh(
    core_axis_name="c", subcore_axis_name="s", num_cores=1
)


@functools.partial(
    pl.kernel,
    out_type=jax.ShapeDtypeStruct((N,), jnp.float32),
    mesh=_mesh,
    scratch_types=[
        pltpu.VMEM((CHUNK,), jnp.int32),
        pltpu.VMEM((2 * CHUNK,), jnp.float32),
        pltpu.VMEM((CHUNK,), jnp.float32),
        pltpu.VMEM((CHUNK,), jnp.float32),
        pltpu.VMEM((CHUNK,), jnp.float32),
        pltpu.VMEM((TBL,), jnp.float32),
        pltpu.VMEM((W, SLICE), jnp.float32),
        pltpu.VMEM((SLICE,), jnp.float32),
        pltpu.VMEM((M,), jnp.float32),
        pltpu.VMEM_SHARED((W, TBL), jnp.float32),
        pltpu.VMEM_SHARED((TBL,), jnp.float32),
    ],
    compiler_params=pltpu.CompilerParams(needs_layout_passes=False),
)
def _ce_kernel(ids_hbm, h_hbm, q_hbm, out_hbm,
               ids_v, h_v, q_v, sinv_v, out_v, tbl_v, part_v, comb_v, r_v,
               sh_all, sh_comb):
    w = lax.axis_index("s")
    start = BASE * w + 8 * jnp.maximum(w - 12, 0)
    cnt = jnp.where(w >= 12, CHUNK, BASE)

    pltpu.sync_copy(ids_hbm.at[pl.ds(start, CHUNK)], ids_v)
    pltpu.sync_copy(h_hbm.at[pl.ds(2 * start, 2 * CHUNK)], h_v)
    pltpu.sync_copy(q_hbm.at[pl.ds(start, CHUNK)], q_v)

    lanes = lax.iota(jnp.int32, L)
    zero = jnp.zeros((L,), jnp.float32)

    def zero_body(k, c):
        tbl_v[pl.ds(k * L, L)] = zero
        return c

    lax.fori_loop(0, TBL // L, zero_body, 0)

    def p1(j, c):
        base = j * L
        ii2 = 2 * base + 2 * lanes
        idx = ids_v[pl.ds(base, L)]
        e = plsc.load_gather(h_v, [ii2])
        s = plsc.load_gather(h_v, [ii2 + 1])
        qv = q_v[pl.ds(base, L)]
        sinv = 1.0 / s
        esinv = e * sinv
        sinv_v[pl.ds(base, L)] = sinv
        m = (base + lanes) < cnt
        plsc.addupdate_scatter(tbl_v, [idx], qv, mask=m)
        plsc.addupdate_scatter(tbl_v, [idx + M], sinv, mask=m)
        plsc.addupdate_scatter(tbl_v, [idx + 2 * M], esinv, mask=m)
        return c

    lax.fori_loop(0, NV, p1, 0)

    pltpu.sync_copy(tbl_v, sh_all.at[w])
    plsc.subcore_barrier()
    pltpu.sync_copy(sh_all.at[:, pl.ds(SLICE * w, SLICE)], part_v)

    def red_body(r, c):
        o = r * L
        acc = part_v[0, pl.ds(o, L)]
        for t in range(1, W):
            acc = acc + part_v[t, pl.ds(o, L)]
        comb_v[pl.ds(o, L)] = acc
        return c

    lax.fori_loop(0, SLICE // L, red_body, 0)
    pltpu.sync_copy(comb_v, sh_comb.at[pl.ds(SLICE * w, SLICE)])
    plsc.subcore_barrier()
    pltpu.sync_copy(sh_comb, tbl_v)

    def rt(k, c):
        o = k * L
        tq = tbl_v[pl.ds(o, L)]
        tsinv = tbl_v[pl.ds(M + o, L)]
        tesinv = tbl_v[pl.ds(2 * M + o, L)]
        r_v[pl.ds(o, L)] = (tq + tesinv) / tsinv
        return c

    lax.fori_loop(0, M // L, rt, 0)

    def p2(j, c):
        base = j * L
        ii2 = 2 * base + 2 * lanes
        idx = ids_v[pl.ds(base, L)]
        e = plsc.load_gather(h_v, [ii2])
        sinv = sinv_v[pl.ds(base, L)]
        r = plsc.load_gather(r_v, [idx])
        out_v[pl.ds(base, L)] = sinv * (r - e)
        return c

    lax.fori_loop(0, NV, p2, 0)

    pltpu.sync_copy(out_v, out_hbm.at[pl.ds(start, CHUNK)])


def kernel(h, q, mol_ids):
    ids32 = mol_ids.astype(jnp.int32)
    return _ce_kernel(ids32, h.reshape(-1), q)


# speedup vs baseline: 31.6086x; 2.0307x over previous
import functools

import jax
import jax.numpy as jnp
from jax import lax
from jax.experimental import pallas as pl
from jax.experimental.pallas import tpu as pltpu
from jax.experimental.pallas import tpu_sc as plsc

N = 100000
M = 4096
W = 16
L = 16
CHUNK = 6256
BASE = 6248
NV = CHUNK // L
TBL = 3 * M
SLICE = TBL // W

_mesh = plsc.VectorSubcoreMesh(
    core_axis_name="c", subcore_axis_name="s", num_cores=1
)


@functools.partial(
    pl.kernel,
    out_type=jax.ShapeDtypeStruct((N,), jnp.float32),
    mesh=_mesh,
    scratch_types=[
        pltpu.VMEM((CHUNK,), jnp.int32),
        pltpu.VMEM((CHUNK,), jnp.float32),
        pltpu.VMEM((CHUNK,), jnp.float32),
        pltpu.VMEM((CHUNK,), jnp.float32),
        pltpu.VMEM((CHUNK,), jnp.float32),
        pltpu.VMEM((CHUNK,), jnp.float32),
        pltpu.VMEM((TBL,), jnp.float32),
        pltpu.VMEM((W, SLICE), jnp.float32),
        pltpu.VMEM((SLICE,), jnp.float32),
        pltpu.VMEM((M,), jnp.float32),
        pltpu.VMEM_SHARED((W, TBL), jnp.float32),
        pltpu.VMEM_SHARED((TBL,), jnp.float32),
    ],
    compiler_params=pltpu.CompilerParams(needs_layout_passes=False),
)
def _ce_kernel(ids_hbm, e_hbm, s_hbm, q_hbm, out_hbm,
               ids_v, e_v, s_v, q_v, sinv_v, out_v, tbl_v, part_v, comb_v,
               r_v, sh_all, sh_comb):
    w = lax.axis_index("s")
    start = BASE * w + 8 * jnp.maximum(w - 12, 0)
    cnt = jnp.where(w >= 12, CHUNK, BASE)

    pltpu.sync_copy(ids_hbm.at[pl.ds(start, CHUNK)], ids_v)
    pltpu.sync_copy(e_hbm.at[pl.ds(start, CHUNK)], e_v)
    pltpu.sync_copy(s_hbm.at[pl.ds(start, CHUNK)], s_v)
    pltpu.sync_copy(q_hbm.at[pl.ds(start, CHUNK)], q_v)

    lanes = lax.iota(jnp.int32, L)
    zero = jnp.zeros((L,), jnp.float32)

    def zero_body(k, c):
        tbl_v[pl.ds(k * L, L)] = zero
        return c

    lax.fori_loop(0, TBL // L, zero_body, 0)

    def p1(j, c):
        base = j * L
        idx = ids_v[pl.ds(base, L)]
        e = e_v[pl.ds(base, L)]
        s = s_v[pl.ds(base, L)]
        qv = q_v[pl.ds(base, L)]
        sinv = 1.0 / s
        esinv = e * sinv
        sinv_v[pl.ds(base, L)] = sinv
        m = (base + lanes) < cnt
        plsc.addupdate_scatter(tbl_v, [idx], qv, mask=m)
        plsc.addupdate_scatter(tbl_v, [idx + M], sinv, mask=m)
        plsc.addupdate_scatter(tbl_v, [idx + 2 * M], esinv, mask=m)
        return c

    lax.fori_loop(0, NV, p1, 0)

    pltpu.sync_copy(tbl_v, sh_all.at[w])
    plsc.subcore_barrier()
    pltpu.sync_copy(sh_all.at[:, pl.ds(SLICE * w, SLICE)], part_v)

    def red_body(r, c):
        o = r * L
        acc = part_v[0, pl.ds(o, L)]
        for t in range(1, W):
            acc = acc + part_v[t, pl.ds(o, L)]
        comb_v[pl.ds(o, L)] = acc
        return c

    lax.fori_loop(0, SLICE // L, red_body, 0)
    pltpu.sync_copy(comb_v, sh_comb.at[pl.ds(SLICE * w, SLICE)])
    plsc.subcore_barrier()
    pltpu.sync_copy(sh_comb, tbl_v)

    def rt(k, c):
        o = k * L
        tq = tbl_v[pl.ds(o, L)]
        tsinv = tbl_v[pl.ds(M + o, L)]
        tesinv = tbl_v[pl.ds(2 * M + o, L)]
        r_v[pl.ds(o, L)] = (tq + tesinv) / tsinv
        return c

    lax.fori_loop(0, M // L, rt, 0)

    def p2(j, c):
        base = j * L
        idx = ids_v[pl.ds(base, L)]
        e = e_v[pl.ds(base, L)]
        sinv = sinv_v[pl.ds(base, L)]
        r = plsc.load_gather(r_v, [idx])
        out_v[pl.ds(base, L)] = sinv * (r - e)
        return c

    lax.fori_loop(0, NV, p2, 0)

    pltpu.sync_copy(out_v, out_hbm.at[pl.ds(start, CHUNK)])


def kernel(h, q, mol_ids):
    ids32 = mol_ids.astype(jnp.int32)
    return _ce_kernel(ids32, h[:, 0], h[:, 1], q)


# speedup vs baseline: 39.5626x; 1.2516x over previous
import functools

import jax
import jax.numpy as jnp
from jax import lax
from jax.experimental import pallas as pl
from jax.experimental.pallas import tpu as pltpu
from jax.experimental.pallas import tpu_sc as plsc

N = 100000
M = 4096
W = 16
L = 16
CHUNK = 6256
BASE = 6248
NV = CHUNK // L
NPAIR = 195
TBL = 2 * M
SLICE = TBL // W

_mesh = plsc.VectorSubcoreMesh(
    core_axis_name="c", subcore_axis_name="s", num_cores=1
)


@functools.partial(
    pl.kernel,
    out_type=jax.ShapeDtypeStruct((N,), jnp.float32),
    mesh=_mesh,
    scratch_types=[
        pltpu.VMEM((CHUNK,), jnp.int32),
        pltpu.VMEM((CHUNK,), jnp.float32),
        pltpu.VMEM((CHUNK,), jnp.float32),
        pltpu.VMEM((CHUNK,), jnp.float32),
        pltpu.VMEM((CHUNK,), jnp.float32),
        pltpu.VMEM((CHUNK,), jnp.float32),
        pltpu.VMEM((TBL,), jnp.float32),
        pltpu.VMEM((W, SLICE), jnp.float32),
        pltpu.VMEM((SLICE,), jnp.float32),
        pltpu.VMEM((M,), jnp.float32),
        pltpu.VMEM_SHARED((W, TBL), jnp.float32),
        pltpu.VMEM_SHARED((TBL,), jnp.float32),
        pltpu.SemaphoreType.DMA,
    ],
    compiler_params=pltpu.CompilerParams(needs_layout_passes=False),
)
def _ce_kernel(ids_hbm, e_hbm, s_hbm, q_hbm, out_hbm,
               ids_v, e_v, s_v, q_v, sinv_v, out_v, tbl_v, part_v, comb_v,
               r_v, sh_all, sh_comb, dsem):
    w = lax.axis_index("s")
    start = BASE * w + 8 * jnp.maximum(w - 12, 0)
    cnt = jnp.where(w >= 12, CHUNK, BASE)

    c1 = pltpu.async_copy(ids_hbm.at[pl.ds(start, CHUNK)], ids_v, dsem)
    c2 = pltpu.async_copy(e_hbm.at[pl.ds(start, CHUNK)], e_v, dsem)
    c3 = pltpu.async_copy(s_hbm.at[pl.ds(start, CHUNK)], s_v, dsem)
    c4 = pltpu.async_copy(q_hbm.at[pl.ds(start, CHUNK)], q_v, dsem)

    lanes = lax.iota(jnp.int32, L)
    zero = jnp.zeros((L,), jnp.float32)

    def zero_body(k, c):
        o = k * (8 * L)
        for u in range(8):
            tbl_v[pl.ds(o + u * L, L)] = zero
        return c

    lax.fori_loop(0, TBL // (8 * L), zero_body, 0)

    c1.wait()
    c2.wait()
    c3.wait()
    c4.wait()

    def p1_one(base, m):
        idx = ids_v[pl.ds(base, L)]
        e = e_v[pl.ds(base, L)]
        s = s_v[pl.ds(base, L)]
        qv = q_v[pl.ds(base, L)]
        sinv = 1.0 / s
        a = qv + e * sinv
        sinv_v[pl.ds(base, L)] = sinv
        plsc.addupdate_scatter(tbl_v, [idx], a, mask=m)
        plsc.addupdate_scatter(tbl_v, [idx + M], sinv, mask=m)

    def p1(j, c):
        base = j * (2 * L)
        p1_one(base, None)
        p1_one(base + L, None)
        return c

    lax.fori_loop(0, NPAIR, p1, 0)
    p1_one(NPAIR * 2 * L, lanes < (cnt - NPAIR * 2 * L))

    pltpu.sync_copy(tbl_v, sh_all.at[w])
    plsc.subcore_barrier()
    pltpu.sync_copy(sh_all.at[:, pl.ds(SLICE * w, SLICE)], part_v)

    def red_body(r, c):
        o = r * L
        acc = part_v[0, pl.ds(o, L)]
        for t in range(1, W):
            acc = acc + part_v[t, pl.ds(o, L)]
        comb_v[pl.ds(o, L)] = acc
        return c

    lax.fori_loop(0, SLICE // L, red_body, 0)
    pltpu.sync_copy(comb_v, sh_comb.at[pl.ds(SLICE * w, SLICE)])
    plsc.subcore_barrier()
    pltpu.sync_copy(sh_comb, tbl_v)

    def rt(k, c):
        o = k * (2 * L)
        for u in range(2):
            ta = tbl_v[pl.ds(o + u * L, L)]
            tb = tbl_v[pl.ds(M + o + u * L, L)]
            r_v[pl.ds(o + u * L, L)] = ta / tb
        return c

    lax.fori_loop(0, M // (2 * L), rt, 0)

    def p2_one(base):
        idx = ids_v[pl.ds(base, L)]
        e = e_v[pl.ds(base, L)]
        sinv = sinv_v[pl.ds(base, L)]
        r = plsc.load_gather(r_v, [idx])
        out_v[pl.ds(base, L)] = sinv * (r - e)

    def p2(j, c):
        base = j * (2 * L)
        p2_one(base)
        p2_one(base + L)
        return c

    lax.fori_loop(0, NPAIR, p2, 0)
    p2_one(NPAIR * 2 * L)

    pltpu.sync_copy(out_v, out_hbm.at[pl.ds(start, CHUNK)])


def kernel(h, q, mol_ids):
    ids32 = mol_ids.astype(jnp.int32)
    return _ce_kernel(ids32, h[:, 0], h[:, 1], q)


# speedup vs baseline: 48.0738x; 1.2151x over previous
import functools

import jax
import jax.numpy as jnp
from jax import lax
from jax.experimental import pallas as pl
from jax.experimental.pallas import tpu as pltpu
from jax.experimental.pallas import tpu_sc as plsc

N = 100000
M = 4096
W = 16
L = 16
CHUNK = 6256
BASE = 6248
NV = CHUNK // L
NPAIR = 195
TBL = 2 * M
SLICE = TBL // W

_mesh = plsc.VectorSubcoreMesh(
    core_axis_name="c", subcore_axis_name="s", num_cores=1
)


@functools.partial(
    pl.kernel,
    out_type=jax.ShapeDtypeStruct((N,), jnp.float32),
    mesh=_mesh,
    scratch_types=[
        pltpu.VMEM((CHUNK,), jnp.int32),
        pltpu.VMEM((CHUNK,), jnp.float32),
        pltpu.VMEM((CHUNK,), jnp.float32),
        pltpu.VMEM((CHUNK,), jnp.float32),
        pltpu.VMEM((CHUNK,), jnp.float32),
        pltpu.VMEM((CHUNK,), jnp.float32),
        pltpu.VMEM((TBL,), jnp.float32),
        pltpu.VMEM((W, SLICE), jnp.float32),
        pltpu.VMEM((SLICE,), jnp.float32),
        pltpu.VMEM((M,), jnp.float32),
        pltpu.VMEM_SHARED((W, TBL), jnp.float32),
        pltpu.VMEM_SHARED((TBL,), jnp.float32),
        pltpu.SemaphoreType.DMA,
    ],
    compiler_params=pltpu.CompilerParams(needs_layout_passes=False),
)
def _ce_kernel(ids_hbm, e_hbm, s_hbm, q_hbm, out_hbm,
               ids_v, e_v, s_v, q_v, sinv_v, out_v, tbl_v, part_v, comb_v,
               r_v, sh_all, sh_comb, dsem):
    w = lax.axis_index("s")
    start = BASE * w + 8 * jnp.maximum(w - 12, 0)
    cnt = jnp.where(w >= 12, CHUNK, BASE)

    c1 = pltpu.async_copy(ids_hbm.at[pl.ds(start, CHUNK)], ids_v, dsem)
    c2 = pltpu.async_copy(e_hbm.at[pl.ds(start, CHUNK)], e_v, dsem)
    c3 = pltpu.async_copy(s_hbm.at[pl.ds(start, CHUNK)], s_v, dsem)
    c4 = pltpu.async_copy(q_hbm.at[pl.ds(start, CHUNK)], q_v, dsem)

    lanes = lax.iota(jnp.int32, L)
    zero = jnp.zeros((L,), jnp.float32)

    @plsc.parallel_loop(0, TBL, step=L, unroll=8)
    def _zero(o):
        tbl_v[pl.ds(o, L)] = zero

    c1.wait()
    c2.wait()
    c3.wait()
    c4.wait()

    def p1_one(base, m):
        idx = ids_v[pl.ds(base, L)]
        e = e_v[pl.ds(base, L)]
        s = s_v[pl.ds(base, L)]
        qv = q_v[pl.ds(base, L)]
        sinv = 1.0 / s
        a = qv + e * sinv
        sinv_v[pl.ds(base, L)] = sinv
        plsc.addupdate_scatter(tbl_v, [idx], a, mask=m)
        plsc.addupdate_scatter(tbl_v, [idx + M], sinv, mask=m)

    @plsc.parallel_loop(0, NPAIR * 2 * L, step=L, unroll=6)
    def _p1(base):
        p1_one(base, None)

    p1_one(NPAIR * 2 * L, lanes < (cnt - NPAIR * 2 * L))

    pltpu.sync_copy(tbl_v, sh_all.at[w])
    plsc.subcore_barrier()
    pltpu.sync_copy(sh_all.at[:, pl.ds(SLICE * w, SLICE)], part_v)

    @plsc.parallel_loop(0, SLICE, step=L, unroll=2)
    def _red(o):
        acc = part_v[0, pl.ds(o, L)]
        for t in range(1, W):
            acc = acc + part_v[t, pl.ds(o, L)]
        comb_v[pl.ds(o, L)] = acc
    pltpu.sync_copy(comb_v, sh_comb.at[pl.ds(SLICE * w, SLICE)])
    plsc.subcore_barrier()
    pltpu.sync_copy(sh_comb, tbl_v)

    @plsc.parallel_loop(0, M, step=L, unroll=8)
    def _rt(o):
        ta = tbl_v[pl.ds(o, L)]
        tb = tbl_v[pl.ds(M + o, L)]
        r_v[pl.ds(o, L)] = ta / tb

    def p2_one(base):
        idx = ids_v[pl.ds(base, L)]
        e = e_v[pl.ds(base, L)]
        sinv = sinv_v[pl.ds(base, L)]
        r = plsc.load_gather(r_v, [idx])
        out_v[pl.ds(base, L)] = sinv * (r - e)

    @plsc.parallel_loop(0, NPAIR * 2 * L, step=L, unroll=6)
    def _p2(base):
        p2_one(base)

    p2_one(NPAIR * 2 * L)

    pltpu.sync_copy(out_v, out_hbm.at[pl.ds(start, CHUNK)])


def kernel(h, q, mol_ids):
    ids32 = mol_ids.astype(jnp.int32)
    return _ce_kernel(ids32, h[:, 0], h[:, 1], q)


# speedup vs baseline: 70.8060x; 1.4729x over previous
import functools

import jax
import jax.numpy as jnp
from jax import lax
from jax.experimental import pallas as pl
from jax.experimental.pallas import tpu as pltpu
from jax.experimental.pallas import tpu_sc as plsc

N = 100000
M = 4096
L = 16
CHUNK = 6256
BASE = 6248
NPAIR = 195
TBL = 2 * M

_mesh = plsc.VectorSubcoreMesh(
    core_axis_name="c", subcore_axis_name="s", num_cores=1
)


@functools.partial(
    pl.kernel,
    out_type=jax.ShapeDtypeStruct((N,), jnp.float32),
    mesh=_mesh,
    scratch_types=[
        pltpu.VMEM((8 + CHUNK,), jnp.int32),
        pltpu.VMEM((CHUNK,), jnp.float32),
        pltpu.VMEM((CHUNK,), jnp.float32),
        pltpu.VMEM((CHUNK,), jnp.float32),
        pltpu.VMEM((CHUNK,), jnp.float32),
        pltpu.VMEM((CHUNK,), jnp.float32),
        pltpu.VMEM((TBL // 128, 128), jnp.float32),
        pltpu.VMEM((TBL // 128,), jnp.int32),
        pltpu.VMEM((M,), jnp.float32),
        pltpu.VMEM_SHARED((TBL // 128, 128), jnp.float32),
        pltpu.SemaphoreType.DMA,
    ],
    compiler_params=pltpu.CompilerParams(needs_layout_passes=False,
                                         skip_device_barrier=True),
)
def _ce_kernel(ids_hbm, es_hbm, q_hbm, out_hbm,
               ids_v, e_v, s_v, q_v, sinv_v, out_v, tbl2_v, idx_v,
               r_v, sh_comb, dsem):
    w = lax.axis_index("s")
    start = BASE * w + 8 * jnp.maximum(w - 12, 0)
    cnt = jnp.where(w >= 12, CHUNK, BASE)

    c1 = pltpu.async_copy(ids_hbm.at[pl.ds(start, CHUNK)],
                          ids_v.at[pl.ds(8, CHUNK)], dsem)
    c2 = pltpu.async_copy(es_hbm.at[pl.ds(start, CHUNK)], e_v, dsem)
    c3 = pltpu.async_copy(es_hbm.at[pl.ds(N + start, CHUNK)], s_v, dsem)
    c4 = pltpu.async_copy(q_hbm.at[pl.ds(start, CHUNK)], q_v, dsem)

    lanes = lax.iota(jnp.int32, L)
    zero = jnp.zeros((L,), jnp.float32)

    @plsc.parallel_loop(0, TBL // 128, step=1, unroll=2)
    def _zero(row):
        for u in range(8):
            tbl2_v[row, pl.ds(u * L, L)] = zero

    @plsc.parallel_loop(0, TBL // 128, step=L)
    def _iv(o):
        idx_v[pl.ds(o, L)] = o + lanes

    @pl.when(w == 0)
    def _():
        pltpu.sync_copy(tbl2_v, sh_comb)

    plsc.subcore_barrier()

    c1.wait()
    c2.wait()
    c3.wait()
    c4.wait()

    def scat(idx, a, sinv, m):
        row = jax.lax.shift_right_logical(idx, 7)
        col = jax.lax.bitwise_and(idx, 127)
        plsc.addupdate_scatter(tbl2_v, [row, col], a, mask=m)
        plsc.addupdate_scatter(tbl2_v, [row + (M // 128), col], sinv, mask=m)

    m0 = lanes == 0
    m15 = lanes == 15

    def p1_tail(base, m):
        idx = ids_v[pl.ds(8 + base, L)]
        e = e_v[pl.ds(base, L)]
        s = s_v[pl.ds(base, L)]
        qv = q_v[pl.ds(base, L)]
        sinv = 1.0 / s
        a = qv + e * sinv
        sinv_v[pl.ds(base, L)] = sinv
        scat(idx, a, sinv, m)

    def p1_one(base):
        idx = ids_v[pl.ds(8 + base, L)]
        idxp = ids_v[pl.ds(7 + base, L)]
        idxn = ids_v[pl.ds(9 + base, L)]
        e = e_v[pl.ds(base, L)]
        s = s_v[pl.ds(base, L)]
        qv = q_v[pl.ds(base, L)]
        sinv = 1.0 / s
        a = qv + e * sinv
        sinv_v[pl.ds(base, L)] = sinv
        first = (idx != idxp) | m0
        last = (idx != idxn) | m15
        ca = plsc.cumsum(a)
        cs = plsc.cumsum(sinv)
        s_idx = plsc.cummax(jnp.where(first, lanes, 0))
        run_a = ca - jnp.take_along_axis(ca - a, s_idx, axis=0)
        run_s = cs - jnp.take_along_axis(cs - sinv, s_idx, axis=0)
        scat(idx, run_a, run_s, last)

    @plsc.parallel_loop(0, NPAIR * 2 * L, step=L, unroll=6)
    def _p1(base):
        p1_one(base)

    p1_tail(NPAIR * 2 * L, lanes < (cnt - NPAIR * 2 * L))

    pltpu.sync_copy(tbl2_v, sh_comb.at[idx_v], add=True)
    plsc.subcore_barrier()
    pltpu.sync_copy(sh_comb, tbl2_v)

    @plsc.parallel_loop(0, M // 128, step=1, unroll=2)
    def _rt(row):
        for u in range(8):
            ta = tbl2_v[row, pl.ds(u * L, L)]
            tb = tbl2_v[row + (M // 128), pl.ds(u * L, L)]
            r_v[pl.ds(row * 128 + u * L, L)] = ta / tb

    def p2_one(base):
        idx = ids_v[pl.ds(8 + base, L)]
        e = e_v[pl.ds(base, L)]
        sinv = sinv_v[pl.ds(base, L)]
        r = plsc.load_gather(r_v, [idx])
        out_v[pl.ds(base, L)] = sinv * (r - e)

    @plsc.parallel_loop(0, NPAIR * 2 * L, step=L, unroll=6)
    def _p2(base):
        p2_one(base)

    p2_one(NPAIR * 2 * L)

    pltpu.sync_copy(out_v, out_hbm.at[pl.ds(start, CHUNK)])


def kernel(h, q, mol_ids):
    ids32 = mol_ids.astype(jnp.int32)
    es = h.T.reshape(-1)
    return _ce_kernel(ids32, es, q)
